# baseline (speedup 1.0000x reference)
import functools

import jax
import jax.numpy as jnp
from jax import lax
from jax.experimental import pallas as pl
from jax.experimental.pallas import tpu as pltpu
from jax.experimental.pallas import tpu_sc as plsc

N = 10000
E = 320000
D = 128
HD = 64
OUT = 40
NTILES = 32
EPT = E // NTILES
EPS = E // 16
CH = 80
NCH = EPS // CH
NB = 5
BS = 25
NBLK = NCH // BS
ROWS_PT = N // 16


def _vector_mesh():
    return plsc.VectorSubcoreMesh(core_axis_name="c", subcore_axis_name="s")


def _sc_degree(dst2d):
    @functools.partial(
        pl.kernel,
        out_type=jax.ShapeDtypeStruct((NTILES, N), jnp.float32),
        mesh=_vector_mesh(),
        scratch_types=[
            pltpu.VMEM((EPT,), jnp.int32),
            pltpu.VMEM((N,), jnp.float32),
        ],
        compiler_params=pltpu.CompilerParams(needs_layout_passes=False,
                                             use_tc_tiling_on_sc=False),
    )
    def deg_kernel(dst_hbm, out_hbm, idx_v, hist_v):
        wid = lax.axis_index("s") * 2 + lax.axis_index("c")
        zeros16 = jnp.zeros((16,), jnp.float32)

        @pl.loop(0, N, step=16)
        def _(i):
            hist_v[pl.ds(i, 16)] = zeros16

        pltpu.sync_copy(dst_hbm.at[wid], idx_v)
        ones16 = jnp.ones((16,), jnp.float32)

        @pl.loop(0, EPT, step=16)
        def _(i):
            plsc.addupdate_scatter(hist_v, [idx_v[pl.ds(i, 16)]], ones16)

        pltpu.sync_copy(hist_v, out_hbm.at[wid])

    return deg_kernel(dst2d)


def _sc_aggregate(y2, src4d, dst4d):
    @functools.partial(
        pl.kernel,
        out_type=jax.ShapeDtypeStruct((2, 16, ROWS_PT, HD), jnp.float32),
        mesh=_vector_mesh(),
        scratch_types=[
            pltpu.VMEM((2, BS, CH), jnp.int32),
            pltpu.VMEM((2, BS, CH), jnp.int32),
            pltpu.VMEM((NB, CH, HD), jnp.float32),
            pltpu.VMEM_SHARED((N, HD), jnp.float32),
            pltpu.VMEM_SHARED((N, HD), jnp.float32),
            pltpu.SemaphoreType.DMA((NB,)),
            pltpu.SemaphoreType.DMA((NB,)),
            pltpu.SemaphoreType.DMA((2,)),
            pltpu.SemaphoreType.DMA((2,)),
        ],
        compiler_params=pltpu.CompilerParams(needs_layout_passes=False,
                                             use_tc_tiling_on_sc=False),
    )
    def agg_kernel(y_hbm, src_hbm, dst_hbm, out_hbm,
                   sbuf, dbuf, gbuf, acc, ystage, gsem, ssem, issem, idsem):
        cid = lax.axis_index("c")
        sid = lax.axis_index("s")
        rows = pl.ds(sid * ROWS_PT, ROWS_PT)

        pltpu.async_copy(y_hbm.at[cid].at[rows], ystage.at[rows], gsem.at[0])
        pltpu.async_copy(y_hbm.at[cid].at[rows], acc.at[rows], gsem.at[1])

        for par in range(2):
            pltpu.async_copy(src_hbm.at[sid, par], sbuf.at[par], issem.at[par])
            pltpu.async_copy(dst_hbm.at[sid, par], dbuf.at[par], idsem.at[par])

        pltpu.make_async_copy(y_hbm.at[cid].at[rows], ystage.at[rows],
                              gsem.at[0]).wait()
        pltpu.make_async_copy(y_hbm.at[cid].at[rows], acc.at[rows],
                              gsem.at[1]).wait()
        plsc.subcore_barrier()

        def process_block(blk, par, prefetch):
            sb = sbuf.at[par]
            db = dbuf.at[par]
            pltpu.make_async_copy(src_hbm.at[sid, blk], sb,
                                  issem.at[par]).wait()
            pltpu.make_async_copy(dst_hbm.at[sid, blk], db,
                                  idsem.at[par]).wait()

            for b in range(NB):
                pltpu.async_copy(ystage.at[sb.at[b]], gbuf.at[b], gsem.at[b])

            @pl.loop(0, BS - NB, step=NB)
            def _(i):
                for b in range(NB):
                    pltpu.make_async_copy(ystage.at[sb.at[i + b]], gbuf.at[b],
                                          gsem.at[b]).wait()
                    pltpu.async_copy(gbuf.at[b], acc.at[db.at[i + b]],
                                     ssem.at[b], add=True)
                for b in range(NB):
                    pltpu.make_async_copy(gbuf.at[b], acc.at[db.at[i + b]],
                                          ssem.at[b]).wait()
                    pltpu.async_copy(ystage.at[sb.at[i + NB + b]], gbuf.at[b],
                                     gsem.at[b])

            for b in range(NB):
                pltpu.make_async_copy(ystage.at[sb.at[BS - NB + b]],
                                      gbuf.at[b], gsem.at[b]).wait()
                pltpu.async_copy(gbuf.at[b], acc.at[db.at[BS - NB + b]],
                                 ssem.at[b], add=True)
            for b in range(NB):
                pltpu.make_async_copy(gbuf.at[b], acc.at[db.at[BS - NB + b]],
                                      ssem.at[b]).wait()

            if prefetch:
                pltpu.async_copy(src_hbm.at[sid, blk + 2], sb, issem.at[par])
                pltpu.async_copy(dst_hbm.at[sid, blk + 2], db, idsem.at[par])

        @pl.loop(0, NBLK - 2, step=2)
        def _(blk):
            process_block(blk, 0, True)
            process_block(blk + 1, 1, True)

        process_block(NBLK - 2, 0, False)
        process_block(NBLK - 1, 1, False)

        plsc.subcore_barrier()
        pltpu.sync_copy(acc.at[rows], out_hbm.at[cid, sid])

    return agg_kernel(y2, src4d, dst4d)


def _tc_pre(x, wgT, degs):
    B = 1000

    def body(x_ref, w_ref, dg_ref, y_ref):
        deg = jnp.sum(dg_ref[...], axis=1, keepdims=True) + 1.0
        dinv = lax.rsqrt(jnp.maximum(deg, 1.0))
        xw = jnp.dot(x_ref[...], w_ref[...], preferred_element_type=jnp.float32)
        y = xw * dinv
        y_ref[0] = y[:, :HD]
        y_ref[1] = y[:, HD:]

    return pl.pallas_call(
        body,
        grid=(N // B,),
        in_specs=[
            pl.BlockSpec((B, D), lambda i: (i, 0)),
            pl.BlockSpec((D, D), lambda i: (0, 0)),
            pl.BlockSpec((B, NTILES), lambda i: (i, 0)),
        ],
        out_specs=pl.BlockSpec((2, B, HD), lambda i: (0, i, 0)),
        out_shape=jax.ShapeDtypeStruct((2, N, HD), jnp.float32),
        compiler_params=pltpu.CompilerParams(
            dimension_semantics=("parallel",)),
    )(x, wgT, degs)


def _tc_post(parts, degs, bg, bwT, swT, wcT, bc):
    B = 1000

    def body(p_ref, dg_ref, bg_ref, bw_ref, sw_ref, wc_ref,
             bc_ref, o_ref):
        deg = jnp.sum(dg_ref[...], axis=1, keepdims=True) + 1.0
        dinv = lax.rsqrt(jnp.maximum(deg, 1.0))
        agg = jnp.concatenate([p_ref[0], p_ref[1]], axis=1)
        h = jnp.maximum(agg * dinv + bg_ref[...], 0.0)

        sig = 1.0 / (1.0 + jnp.exp(-h))
        z = jnp.dot(h * sig, bw_ref[...], preferred_element_type=jnp.float32)

        u = (h + 2.2) * 2.5
        iu = jnp.floor(u)
        t = u - iu
        t2 = t * t
        t3 = t2 * t
        sixth = 1.0 / 6.0
        p0 = t3 * sixth
        p1 = (((-3.0 * t + 3.0) * t + 3.0) * t + 1.0) * sixth
        p2 = ((3.0 * t - 6.0) * t2) * sixth + 4.0 * sixth
        omt = 1.0 - t
        p3 = omt * omt * omt * sixth
        zero = jnp.zeros_like(h)
        segs = [p0, p1, p2, p3]

        spl = None
        for k in range(2, 8):
            bk = jnp.where(iu == k, segs[0], zero)
            for s in range(1, 4):
                bk = bk + jnp.where(iu == (k + s), segs[s], zero)
            d = jnp.dot(bk, sw_ref[k - 2], preferred_element_type=jnp.float32)
            spl = d if spl is None else spl + d

        logits = jnp.dot(z + spl, wc_ref[...],
                         preferred_element_type=jnp.float32) + bc_ref[...]
        m = jnp.max(logits, axis=1, keepdims=True)
        lse = jnp.log(jnp.sum(jnp.exp(logits - m), axis=1, keepdims=True)) + m
        o_ref[...] = logits - lse

    return pl.pallas_call(
        body,
        grid=(N // B,),
        in_specs=[
            pl.BlockSpec((2, B, HD), lambda i: (0, i, 0)),
            pl.BlockSpec((B, NTILES), lambda i: (i, 0)),
            pl.BlockSpec((1, D), lambda i: (0, 0)),
            pl.BlockSpec((D, D), lambda i: (0, 0)),
            pl.BlockSpec((6, D, D), lambda i: (0, 0, 0)),
            pl.BlockSpec((D, OUT), lambda i: (0, 0)),
            pl.BlockSpec((1, OUT), lambda i: (0, 0)),
        ],
        out_specs=pl.BlockSpec((B, OUT), lambda i: (i, 0)),
        out_shape=jax.ShapeDtypeStruct((N, OUT), jnp.float32),
        compiler_params=pltpu.CompilerParams(
            dimension_semantics=("parallel",)),
    )(parts, degs, bg, bwT, swT, wcT, bc)


def kernel(x, w_gcn, b_gcn, base_weight, spline_weight, grid, w_cls, b_cls,
           edge_index):
    edge_index = edge_index.astype(jnp.int32)
    src4 = edge_index[0].reshape(16, NBLK, BS, CH)
    dst2 = edge_index[1].reshape(NTILES, EPT)
    dst4 = edge_index[1].reshape(16, NBLK, BS, CH)

    degs = _sc_degree(dst2).T
    y2 = _tc_pre(x, w_gcn.T, degs)
    parts = _sc_aggregate(y2, src4, dst4)
    parts = parts.reshape(2, N, HD)
    swT = (spline_weight.reshape(D, D * 8).T
           .reshape(D, 8, D).transpose(1, 0, 2)[2:8])
    out = _tc_post(
        parts, degs,
        b_gcn.reshape(1, D),
        base_weight.T,
        swT,
        w_cls.T,
        b_cls.reshape(1, OUT),
    )
    return out


# speedup vs baseline: 1.3185x; 1.3185x over previous
import functools

import jax
import jax.numpy as jnp
from jax import lax
from jax.experimental import pallas as pl
from jax.experimental.pallas import tpu as pltpu
from jax.experimental.pallas import tpu_sc as plsc

N = 10000
E = 320000
D = 128
HD = 64
OUT = 40
NTILES = 32
EPT = E // NTILES
EPS = E // 16
CH = 80
NCH = EPS // CH
NB = 5
BS = 25
NBLK = NCH // BS
ROWS_PT = N // 16


def _vector_mesh():
    return plsc.VectorSubcoreMesh(core_axis_name="c", subcore_axis_name="s")


def _sc_degree(dst2d):
    @functools.partial(
        pl.kernel,
        out_type=jax.ShapeDtypeStruct((NTILES, N), jnp.float32),
        mesh=_vector_mesh(),
        scratch_types=[
            pltpu.VMEM((EPT,), jnp.int32),
            pltpu.VMEM((N,), jnp.float32),
        ],
        compiler_params=pltpu.CompilerParams(needs_layout_passes=False,
                                             use_tc_tiling_on_sc=False),
    )
    def deg_kernel(dst_hbm, out_hbm, idx_v, hist_v):
        wid = lax.axis_index("s") * 2 + lax.axis_index("c")
        zeros16 = jnp.zeros((16,), jnp.float32)

        @pl.loop(0, N, step=16)
        def _(i):
            hist_v[pl.ds(i, 16)] = zeros16

        pltpu.sync_copy(dst_hbm.at[wid], idx_v)
        ones16 = jnp.ones((16,), jnp.float32)

        @pl.loop(0, EPT, step=16)
        def _(i):
            plsc.addupdate_scatter(hist_v, [idx_v[pl.ds(i, 16)]], ones16)

        pltpu.sync_copy(hist_v, out_hbm.at[wid])

    return deg_kernel(dst2d)


def _sc_aggregate(y2, src3d, dst3d):
    @functools.partial(
        pl.kernel,
        out_type=jax.ShapeDtypeStruct((2, 16, ROWS_PT, HD), jnp.float32),
        mesh=_vector_mesh(),
        scratch_types=[
            pltpu.VMEM((NCH, CH), jnp.int32),
            pltpu.VMEM((NCH, CH), jnp.int32),
            pltpu.VMEM((NB, CH, HD), jnp.float32),
            pltpu.VMEM_SHARED((N, HD), jnp.float32),
            pltpu.SemaphoreType.DMA((NB,)),
            pltpu.SemaphoreType.DMA((NB,)),
        ],
        compiler_params=pltpu.CompilerParams(needs_layout_passes=False,
                                             use_tc_tiling_on_sc=False),
    )
    def agg_kernel(y_hbm, src_hbm, dst_hbm, out_hbm,
                   srcv, dstv, gbuf, acc, gsem, ssem):
        cid = lax.axis_index("c")
        sid = lax.axis_index("s")
        rows = pl.ds(sid * ROWS_PT, ROWS_PT)

        pltpu.async_copy(y_hbm.at[cid].at[rows], acc.at[rows], gsem.at[0])
        pltpu.sync_copy(src_hbm.at[sid], srcv)
        pltpu.sync_copy(dst_hbm.at[sid], dstv)
        pltpu.make_async_copy(y_hbm.at[cid].at[rows], acc.at[rows],
                              gsem.at[0]).wait()
        plsc.subcore_barrier()

        yh = y_hbm.at[cid]
        for b in range(NB):
            pltpu.async_copy(yh.at[srcv.at[b]], gbuf.at[b], gsem.at[b])

        @pl.loop(0, NCH - NB, step=NB)
        def _(i):
            for b in range(NB):
                pltpu.make_async_copy(yh.at[srcv.at[i + b]], gbuf.at[b],
                                      gsem.at[b]).wait()
                pltpu.async_copy(gbuf.at[b], acc.at[dstv.at[i + b]],
                                 ssem.at[b], add=True)
            for b in range(NB):
                pltpu.make_async_copy(gbuf.at[b], acc.at[dstv.at[i + b]],
                                      ssem.at[b]).wait()
                pltpu.async_copy(yh.at[srcv.at[i + NB + b]], gbuf.at[b],
                                 gsem.at[b])

        for b in range(NB):
            pltpu.make_async_copy(yh.at[srcv.at[NCH - NB + b]], gbuf.at[b],
                                  gsem.at[b]).wait()
            pltpu.async_copy(gbuf.at[b], acc.at[dstv.at[NCH - NB + b]],
                             ssem.at[b], add=True)
        for b in range(NB):
            pltpu.make_async_copy(gbuf.at[b], acc.at[dstv.at[NCH - NB + b]],
                                  ssem.at[b]).wait()

        plsc.subcore_barrier()
        pltpu.sync_copy(acc.at[rows], out_hbm.at[cid, sid])

    return agg_kernel(y2, src3d, dst3d)


def _tc_pre(x, wgT, degs):
    B = 1000

    def body(x_ref, w_ref, dg_ref, y_ref):
        deg = jnp.sum(dg_ref[...], axis=1, keepdims=True) + 1.0
        dinv = lax.rsqrt(jnp.maximum(deg, 1.0))
        xw = jnp.dot(x_ref[...], w_ref[...], preferred_element_type=jnp.float32)
        y = xw * dinv
        y_ref[0] = y[:, :HD]
        y_ref[1] = y[:, HD:]

    return pl.pallas_call(
        body,
        grid=(N // B,),
        in_specs=[
            pl.BlockSpec((B, D), lambda i: (i, 0)),
            pl.BlockSpec((D, D), lambda i: (0, 0)),
            pl.BlockSpec((B, NTILES), lambda i: (i, 0)),
        ],
        out_specs=pl.BlockSpec((2, B, HD), lambda i: (0, i, 0)),
        out_shape=jax.ShapeDtypeStruct((2, N, HD), jnp.float32),
        compiler_params=pltpu.CompilerParams(
            dimension_semantics=("parallel",)),
    )(x, wgT, degs)


def _tc_post(parts, degs, bg, bwT, swT, wcT, bc):
    B = 1000

    def body(p_ref, dg_ref, bg_ref, bw_ref, sw_ref, wc_ref,
             bc_ref, o_ref):
        deg = jnp.sum(dg_ref[...], axis=1, keepdims=True) + 1.0
        dinv = lax.rsqrt(jnp.maximum(deg, 1.0))
        agg = jnp.concatenate([p_ref[0], p_ref[1]], axis=1)
        h = jnp.maximum(agg * dinv + bg_ref[...], 0.0)

        sig = 1.0 / (1.0 + jnp.exp(-h))
        z = jnp.dot(h * sig, bw_ref[...], preferred_element_type=jnp.float32)

        u = (h + 2.2) * 2.5
        iu = jnp.floor(u)
        t = u - iu
        t2 = t * t
        t3 = t2 * t
        sixth = 1.0 / 6.0
        p0 = t3 * sixth
        p1 = (((-3.0 * t + 3.0) * t + 3.0) * t + 1.0) * sixth
        p2 = ((3.0 * t - 6.0) * t2) * sixth + 4.0 * sixth
        omt = 1.0 - t
        p3 = omt * omt * omt * sixth
        zero = jnp.zeros_like(h)
        segs = [p0, p1, p2, p3]

        spl = None
        for k in range(2, 8):
            bk = jnp.where(iu == k, segs[0], zero)
            for s in range(1, 4):
                bk = bk + jnp.where(iu == (k + s), segs[s], zero)
            d = jnp.dot(bk, sw_ref[k - 2], preferred_element_type=jnp.float32)
            spl = d if spl is None else spl + d

        logits = jnp.dot(z + spl, wc_ref[...],
                         preferred_element_type=jnp.float32) + bc_ref[...]
        m = jnp.max(logits, axis=1, keepdims=True)
        lse = jnp.log(jnp.sum(jnp.exp(logits - m), axis=1, keepdims=True)) + m
        o_ref[...] = logits - lse

    return pl.pallas_call(
        body,
        grid=(N // B,),
        in_specs=[
            pl.BlockSpec((2, B, HD), lambda i: (0, i, 0)),
            pl.BlockSpec((B, NTILES), lambda i: (i, 0)),
            pl.BlockSpec((1, D), lambda i: (0, 0)),
            pl.BlockSpec((D, D), lambda i: (0, 0)),
            pl.BlockSpec((6, D, D), lambda i: (0, 0, 0)),
            pl.BlockSpec((D, OUT), lambda i: (0, 0)),
            pl.BlockSpec((1, OUT), lambda i: (0, 0)),
        ],
        out_specs=pl.BlockSpec((B, OUT), lambda i: (i, 0)),
        out_shape=jax.ShapeDtypeStruct((N, OUT), jnp.float32),
        compiler_params=pltpu.CompilerParams(
            dimension_semantics=("parallel",)),
    )(parts, degs, bg, bwT, swT, wcT, bc)


def kernel(x, w_gcn, b_gcn, base_weight, spline_weight, grid, w_cls, b_cls,
           edge_index):
    edge_index = edge_index.astype(jnp.int32)
    src3 = edge_index[0].reshape(16, NCH, CH)
    dst2 = edge_index[1].reshape(NTILES, EPT)
    dst3 = edge_index[1].reshape(16, NCH, CH)

    degs = _sc_degree(dst2).T
    y2 = _tc_pre(x, w_gcn.T, degs)
    parts = _sc_aggregate(y2, src3, dst3)
    parts = parts.reshape(2, N, HD)
    swT = (spline_weight.reshape(D, D * 8).T
           .reshape(D, 8, D).transpose(1, 0, 2)[2:8])
    out = _tc_post(
        parts, degs,
        b_gcn.reshape(1, D),
        base_weight.T,
        swT,
        w_cls.T,
        b_cls.reshape(1, OUT),
    )
    return out


# speedup vs baseline: 1.3191x; 1.0005x over previous
import functools

import jax
import jax.numpy as jnp
from jax import lax
from jax.experimental import pallas as pl
from jax.experimental.pallas import tpu as pltpu
from jax.experimental.pallas import tpu_sc as plsc

N = 10000
E = 320000
D = 128
HD = 64
OUT = 40
NTILES = 32
EPT = E // NTILES
EPS = E // 16
CH = 80
NCH = EPS // CH
NB = 5
ROWS_PT = N // 16


def _vector_mesh():
    return plsc.VectorSubcoreMesh(core_axis_name="c", subcore_axis_name="s")


def _sc_degree(dst2d):
    @functools.partial(
        pl.kernel,
        out_type=jax.ShapeDtypeStruct((NTILES, N), jnp.float32),
        mesh=_vector_mesh(),
        scratch_types=[
            pltpu.VMEM((EPT,), jnp.int32),
            pltpu.VMEM((N,), jnp.float32),
        ],
        compiler_params=pltpu.CompilerParams(needs_layout_passes=False,
                                             use_tc_tiling_on_sc=False),
    )
    def deg_kernel(dst_hbm, out_hbm, idx_v, hist_v):
        wid = lax.axis_index("s") * 2 + lax.axis_index("c")
        zeros16 = jnp.zeros((16,), jnp.float32)

        @pl.loop(0, N, step=16)
        def _(i):
            hist_v[pl.ds(i, 16)] = zeros16

        pltpu.sync_copy(dst_hbm.at[wid], idx_v)
        ones16 = jnp.ones((16,), jnp.float32)

        @pl.loop(0, EPT, step=16)
        def _(i):
            plsc.addupdate_scatter(hist_v, [idx_v[pl.ds(i, 16)]], ones16)

        pltpu.sync_copy(hist_v, out_hbm.at[wid])

    return deg_kernel(dst2d)


def _sc_aggregate(y2, src3d, dst3d):
    @functools.partial(
        pl.kernel,
        out_type=jax.ShapeDtypeStruct((2, 16, ROWS_PT, HD), jnp.float32),
        mesh=_vector_mesh(),
        scratch_types=[
            pltpu.VMEM((NCH, CH), jnp.int32),
            pltpu.VMEM((NCH, CH), jnp.int32),
            pltpu.VMEM((NB, CH, HD), jnp.float32),
            pltpu.VMEM_SHARED((N, HD), jnp.float32),
            pltpu.SemaphoreType.DMA((NB,)),
            pltpu.SemaphoreType.DMA((NB,)),
        ],
        compiler_params=pltpu.CompilerParams(needs_layout_passes=False,
                                             use_tc_tiling_on_sc=False),
    )
    def agg_kernel(y_hbm, src_hbm, dst_hbm, out_hbm,
                   srcv, dstv, gbuf, acc, gsem, ssem):
        cid = lax.axis_index("c")
        sid = lax.axis_index("s")
        rows = pl.ds(sid * ROWS_PT, ROWS_PT)

        pltpu.async_copy(y_hbm.at[cid].at[rows], acc.at[rows], gsem.at[0])
        pltpu.sync_copy(src_hbm.at[sid], srcv)
        pltpu.sync_copy(dst_hbm.at[sid], dstv)
        pltpu.make_async_copy(y_hbm.at[cid].at[rows], acc.at[rows],
                              gsem.at[0]).wait()
        plsc.subcore_barrier()

        yh = y_hbm.at[cid]
        for b in range(NB):
            pltpu.async_copy(yh.at[srcv.at[b]], gbuf.at[b], gsem.at[b])

        @pl.loop(0, NCH - NB, step=NB)
        def _(i):
            for b in range(NB):
                pltpu.make_async_copy(yh.at[srcv.at[i + b]], gbuf.at[b],
                                      gsem.at[b]).wait()
                pltpu.async_copy(gbuf.at[b], acc.at[dstv.at[i + b]],
                                 ssem.at[b], add=True)
            for b in range(NB):
                pltpu.make_async_copy(gbuf.at[b], acc.at[dstv.at[i + b]],
                                      ssem.at[b]).wait()
                pltpu.async_copy(yh.at[srcv.at[i + NB + b]], gbuf.at[b],
                                 gsem.at[b])

        for b in range(NB):
            pltpu.make_async_copy(yh.at[srcv.at[NCH - NB + b]], gbuf.at[b],
                                  gsem.at[b]).wait()
            pltpu.async_copy(gbuf.at[b], acc.at[dstv.at[NCH - NB + b]],
                             ssem.at[b], add=True)
        for b in range(NB):
            pltpu.make_async_copy(gbuf.at[b], acc.at[dstv.at[NCH - NB + b]],
                                  ssem.at[b]).wait()

        plsc.subcore_barrier()
        pltpu.sync_copy(acc.at[rows], out_hbm.at[cid, sid])

    return agg_kernel(y2, src3d, dst3d)


def _tc_pre(x, wgT, degs):
    B = 1000

    def body(x_ref, w_ref, dg_ref, y_ref):
        deg = jnp.sum(dg_ref[...], axis=1, keepdims=True) + 1.0
        dinv = lax.rsqrt(jnp.maximum(deg, 1.0))
        xw = jnp.dot(x_ref[...], w_ref[...], preferred_element_type=jnp.float32)
        y = xw * dinv
        y_ref[0] = y[:, :HD]
        y_ref[1] = y[:, HD:]

    return pl.pallas_call(
        body,
        grid=(N // B,),
        in_specs=[
            pl.BlockSpec((B, D), lambda i: (i, 0)),
            pl.BlockSpec((D, D), lambda i: (0, 0)),
            pl.BlockSpec((B, NTILES), lambda i: (i, 0)),
        ],
        out_specs=pl.BlockSpec((2, B, HD), lambda i: (0, i, 0)),
        out_shape=jax.ShapeDtypeStruct((2, N, HD), jnp.float32),
        compiler_params=pltpu.CompilerParams(
            dimension_semantics=("parallel",)),
    )(x, wgT, degs)


def _tc_post(parts, degs, bg, bwT, swT, wcT, bc):
    B = 1000

    def body(p_ref, dg_ref, bg_ref, bw_ref, sw_ref, wc_ref,
             bc_ref, o_ref):
        deg = jnp.sum(dg_ref[...], axis=1, keepdims=True) + 1.0
        dinv = lax.rsqrt(jnp.maximum(deg, 1.0))
        agg = jnp.concatenate([p_ref[0], p_ref[1]], axis=1)
        h = jnp.maximum(agg * dinv + bg_ref[...], 0.0)

        sig = 1.0 / (1.0 + jnp.exp(-h))
        z = jnp.dot(h * sig, bw_ref[...], preferred_element_type=jnp.float32)

        u = (h + 2.2) * 2.5
        iu = jnp.floor(u)
        t = u - iu
        t2 = t * t
        t3 = t2 * t
        sixth = 1.0 / 6.0
        p0 = t3 * sixth
        p1 = (((-3.0 * t + 3.0) * t + 3.0) * t + 1.0) * sixth
        p2 = ((3.0 * t - 6.0) * t2) * sixth + 4.0 * sixth
        omt = 1.0 - t
        p3 = omt * omt * omt * sixth
        zero = jnp.zeros_like(h)
        segs = [p0, p1, p2, p3]

        spl = None
        for k in range(2, 8):
            bk = jnp.where(iu == k, segs[0], zero)
            for s in range(1, 4):
                bk = bk + jnp.where(iu == (k + s), segs[s], zero)
            d = jnp.dot(bk, sw_ref[k - 2], preferred_element_type=jnp.float32)
            spl = d if spl is None else spl + d

        logits = jnp.dot(z + spl, wc_ref[...],
                         preferred_element_type=jnp.float32) + bc_ref[...]
        m = jnp.max(logits, axis=1, keepdims=True)
        lse = jnp.log(jnp.sum(jnp.exp(logits - m), axis=1, keepdims=True)) + m
        o_ref[...] = logits - lse

    return pl.pallas_call(
        body,
        grid=(N // B,),
        in_specs=[
            pl.BlockSpec((2, B, HD), lambda i: (0, i, 0)),
            pl.BlockSpec((B, NTILES), lambda i: (i, 0)),
            pl.BlockSpec((1, D), lambda i: (0, 0)),
            pl.BlockSpec((D, D), lambda i: (0, 0)),
            pl.BlockSpec((6, D, D), lambda i: (0, 0, 0)),
            pl.BlockSpec((D, OUT), lambda i: (0, 0)),
            pl.BlockSpec((1, OUT), lambda i: (0, 0)),
        ],
        out_specs=pl.BlockSpec((B, OUT), lambda i: (i, 0)),
        out_shape=jax.ShapeDtypeStruct((N, OUT), jnp.float32),
        compiler_params=pltpu.CompilerParams(
            dimension_semantics=("parallel",)),
    )(parts, degs, bg, bwT, swT, wcT, bc)


def kernel(x, w_gcn, b_gcn, base_weight, spline_weight, grid, w_cls, b_cls,
           edge_index):
    edge_index = edge_index.astype(jnp.int32)
    src3 = edge_index[0].reshape(16, NCH, CH)
    dst2 = edge_index[1].reshape(NTILES, EPT)
    dst3 = edge_index[1].reshape(16, NCH, CH)

    degs = _sc_degree(dst2).T
    y2 = _tc_pre(x, w_gcn.T, degs)
    parts = _sc_aggregate(y2, src3, dst3)
    parts = parts.reshape(2, N, HD)
    swT = (spline_weight.reshape(D, D * 8).T
           .reshape(D, 8, D).transpose(1, 0, 2)[2:8])
    out = _tc_post(
        parts, degs,
        b_gcn.reshape(1, D),
        base_weight.T,
        swT,
        w_cls.T,
        b_cls.reshape(1, OUT),
    )
    return out
